# Initial kernel scaffold; baseline (speedup 1.0000x reference)
import jax
import jax.numpy as jnp
from jax.experimental import pallas as pl


def kernel(x, edge_index, W1, b1, Wc1, bc1, W2, b2, Wc2, bc2, W3, b3, Wc3, bc3, Wf1, bf1, Wf2, bf2):
    raise NotImplementedError("write your pallas kernel here")


# speedup vs baseline: 4.6635x; 4.6635x over previous
import functools

import jax
import jax.numpy as jnp
from jax import lax
from jax.experimental import pallas as pl
from jax.experimental.pallas import tpu as pltpu
from jax.experimental.pallas import tpu_sc as plsc

_N = 10000
_E = 320000
_NC = 2
_NS = 16
_NW = _NC * _NS
_EPW = _E // _NW
_C = 400
_NCH = _EPW // _C
_C1 = 200
_NCH1 = _E // _NS // _C1
_R1 = 632
_R2 = _N - 15 * _R1

_f32 = jnp.float32


def _sc_mesh():
    return plsc.VectorSubcoreMesh(core_axis_name="c", subcore_axis_name="s")


_SC_PARAMS = pltpu.CompilerParams(use_tc_tiling_on_sc=False,
                                  needs_layout_passes=False)


def _tile_rows_copy(sid, copy_fn):
    @pl.when(sid < 15)
    def _():
        copy_fn(pl.multiple_of(sid * _R1, 8), _R1)

    @pl.when(sid == 15)
    def _():
        copy_fn(15 * _R1, _R2)


def _sc_deg_seg(g1a, g1b, src_s, dst_s, z_acc, z_deg, ones_blk):
    d = g1a.shape[1]

    @functools.partial(
        pl.kernel,
        out_type=(
            jax.ShapeDtypeStruct((_NC, _N, d), _f32),
            jax.ShapeDtypeStruct((_N, 16), _f32),
        ),
        mesh=_sc_mesh(),
        compiler_params=_SC_PARAMS,
        scratch_types=[
            pltpu.VMEM((_NCH1, _C1), jnp.int32),
            pltpu.VMEM((_NCH1, _C1), jnp.int32),
            pltpu.VMEM((_C1, d), _f32),
            pltpu.VMEM((_C1, 16), _f32),
            pltpu.VMEM_SHARED((_N, d), _f32),
            pltpu.VMEM_SHARED((_N, 16), _f32),
            pltpu.SemaphoreType.DMA,
        ],
    )
    def k(ga_hbm, gb_hbm, src_hbm, dst_hbm, zacc_hbm, zdeg_hbm, ones_hbm,
          acc_out, deg_out, src_v, dst_v, r_v, ones_v, acc, dacc, sem):
        cid = lax.axis_index("c")
        sid = lax.axis_index("s")
        pltpu.sync_copy(src_hbm.at[sid], src_v)
        pltpu.sync_copy(dst_hbm.at[sid], dst_v)
        pltpu.sync_copy(ones_hbm, ones_v)

        def zero_rows(r0, nr):
            pltpu.sync_copy(zacc_hbm.at[pl.ds(r0, nr)], acc.at[pl.ds(r0, nr)])
            pltpu.sync_copy(zdeg_hbm.at[pl.ds(r0, nr)], dacc.at[pl.ds(r0, nr)])

        _tile_rows_copy(sid, zero_rows)
        plsc.subcore_barrier()

        @pl.when(cid == 0)
        def _():
            def step(j, carry):
                sj = src_v.at[j]
                dj = dst_v.at[j]
                pltpu.async_copy(ga_hbm.at[sj], r_v, sem).wait()
                pltpu.sync_copy(r_v, acc.at[dj], add=True)
                pltpu.sync_copy(ones_v, dacc.at[dj], add=True)
                return carry

            lax.fori_loop(0, _NCH1, step, 0)

        @pl.when(cid == 1)
        def _():
            def step(j, carry):
                sj = src_v.at[j]
                dj = dst_v.at[j]
                pltpu.async_copy(gb_hbm.at[sj], r_v, sem).wait()
                pltpu.sync_copy(r_v, acc.at[dj], add=True)
                return carry

            lax.fori_loop(0, _NCH1, step, 0)

        plsc.subcore_barrier()

        def write_rows(r0, nr):
            pltpu.sync_copy(acc.at[pl.ds(r0, nr)],
                            acc_out.at[cid, pl.ds(r0, nr)])

        _tile_rows_copy(sid, write_rows)

        @pl.when(cid == 0)
        def _():
            def write_deg(r0, nr):
                pltpu.sync_copy(dacc.at[pl.ds(r0, nr)],
                                deg_out.at[pl.ds(r0, nr)])

            _tile_rows_copy(sid, write_deg)

    return k(g1a, g1b, src_s, dst_s, z_acc, z_deg, ones_blk)


def _sc_cond_seg(p, g, src_t, dst_t, z_acc):
    d = p.shape[1]

    @functools.partial(
        pl.kernel,
        out_type=(
            jax.ShapeDtypeStruct((_E, d), _f32),
            jax.ShapeDtypeStruct((_E, d), _f32),
            jax.ShapeDtypeStruct((_NC, _N, d), _f32),
        ),
        mesh=_sc_mesh(),
        compiler_params=_SC_PARAMS,
        scratch_types=[
            pltpu.VMEM((_NCH, _C), jnp.int32),
            pltpu.VMEM((_NCH, _C), jnp.int32),
            pltpu.VMEM((_C, d), _f32),
            pltpu.VMEM((_C, d), _f32),
            pltpu.VMEM_SHARED((_N, d), _f32),
            pltpu.SemaphoreType.DMA,
        ],
    )
    def k(p_hbm, g_hbm, src_hbm, dst_hbm, zacc_hbm,
          es_out, ed_out, acc_out, src_v, dst_v, e_v, r_v, acc, sem):
        cid = lax.axis_index("c")
        sid = lax.axis_index("s")
        wid = sid * _NC + cid
        pltpu.sync_copy(src_hbm.at[wid], src_v)
        pltpu.sync_copy(dst_hbm.at[wid], dst_v)

        def zero_rows(r0, nr):
            pltpu.sync_copy(zacc_hbm.at[pl.ds(r0, nr)], acc.at[pl.ds(r0, nr)])

        _tile_rows_copy(sid, zero_rows)
        plsc.subcore_barrier()

        def step(j, carry):
            sj = src_v.at[j]
            dj = dst_v.at[j]
            base = pl.multiple_of(wid * _EPW + j * _C, 16)
            pltpu.async_copy(p_hbm.at[sj], e_v, sem).wait()
            pltpu.sync_copy(e_v, es_out.at[pl.ds(base, _C)])
            pltpu.async_copy(p_hbm.at[dj], e_v, sem).wait()
            pltpu.sync_copy(e_v, ed_out.at[pl.ds(base, _C)])
            pltpu.async_copy(g_hbm.at[sj], r_v, sem).wait()
            pltpu.sync_copy(r_v, acc.at[dj], add=True)
            return carry

        lax.fori_loop(0, _NCH, step, 0)
        plsc.subcore_barrier()

        def write_rows(r0, nr):
            pltpu.sync_copy(acc.at[pl.ds(r0, nr)],
                            acc_out.at[cid, pl.ds(r0, nr)])

        _tile_rows_copy(sid, write_rows)

    return k(p, g, src_t, dst_t, z_acc)


def _sc_final(p3, t2, src_f, dst_f):
    @functools.partial(
        pl.kernel,
        out_type=jax.ShapeDtypeStruct((_E,), _f32),
        mesh=_sc_mesh(),
        compiler_params=_SC_PARAMS,
        scratch_types=[
            pltpu.VMEM((_N,), _f32),
            pltpu.VMEM((_EPW,), _f32),
            pltpu.VMEM((_EPW,), jnp.int32),
            pltpu.VMEM((_EPW,), jnp.int32),
            pltpu.VMEM((_EPW,), _f32),
        ],
    )
    def k(p3_hbm, t2_hbm, src_hbm, dst_hbm, out_hbm, p3_v, t_v, s_v, d_v, o_v):
        cid = lax.axis_index("c")
        sid = lax.axis_index("s")
        wid = sid * _NC + cid
        e0 = pl.multiple_of(wid * _EPW, 8)
        pltpu.sync_copy(p3_hbm, p3_v)
        pltpu.sync_copy(t2_hbm.at[pl.ds(e0, _EPW)], t_v)
        pltpu.sync_copy(src_hbm.at[pl.ds(e0, _EPW)], s_v)
        pltpu.sync_copy(dst_hbm.at[pl.ds(e0, _EPW)], d_v)

        def step(kk, carry):
            off = pl.multiple_of(kk * 16, 16)
            s16 = s_v[pl.ds(off, 16)]
            d16 = d_v[pl.ds(off, 16)]
            a = plsc.load_gather(p3_v, [s16])
            b = plsc.load_gather(p3_v, [d16])
            z = t_v[pl.ds(off, 16)] + jnp.maximum(a + b, 0.0)
            o_v[pl.ds(off, 16)] = 1.0 / (1.0 + jnp.exp(-z))
            return carry

        lax.fori_loop(0, _EPW // 16, step, 0)
        pltpu.sync_copy(o_v, out_hbm.at[pl.ds(e0, _EPW)])

    return k(p3, t2, src_f, dst_f)


def _tc_mm(x, w, br=1000):
    n, _ = x.shape
    dout = w.shape[1]

    def body(x_ref, w_ref, o_ref):
        o_ref[...] = jnp.dot(x_ref[...], w_ref[...],
                             preferred_element_type=_f32)

    return pl.pallas_call(
        body,
        grid=(n // br,),
        in_specs=[
            pl.BlockSpec((br, x.shape[1]), lambda i: (i, 0)),
            pl.BlockSpec(w.shape, lambda i: (0, 0)),
        ],
        out_specs=pl.BlockSpec((br, dout), lambda i: (i, 0)),
        out_shape=jax.ShapeDtypeStruct((n, dout), _f32),
    )(x, w)


def _tc_node1(ga, gb, aa, ab, cdeg, ba, bb, wpa, wpb, wga, wgb):
    br = 1000
    d = ga.shape[1]

    def body(ga_ref, gb_ref, aa_ref, ab_ref, c_ref, ba_ref, bb_ref,
             wpa_ref, wpb_ref, wga_ref, wgb_ref, p_ref, gn_ref, dinv_ref):
        cnt = c_ref[:, 0:1]
        dinv = 1.0 / jnp.maximum(cnt, 1.0)
        ha = jnp.maximum(ga_ref[...] + aa_ref[...] * dinv + ba_ref[...], 0.0)
        hb = jnp.maximum(gb_ref[...] + ab_ref[...] * dinv + bb_ref[...], 0.0)
        p_ref[...] = (jnp.dot(ha, wpa_ref[...], preferred_element_type=_f32)
                      + jnp.dot(hb, wpb_ref[...], preferred_element_type=_f32))
        gn_ref[...] = (jnp.dot(ha, wga_ref[...], preferred_element_type=_f32)
                       + jnp.dot(hb, wgb_ref[...], preferred_element_type=_f32))
        dinv_ref[...] = dinv

    return pl.pallas_call(
        body,
        grid=(_N // br,),
        in_specs=[
            pl.BlockSpec((br, d), lambda i: (i, 0)),
            pl.BlockSpec((br, d), lambda i: (i, 0)),
            pl.BlockSpec((br, d), lambda i: (i, 0)),
            pl.BlockSpec((br, d), lambda i: (i, 0)),
            pl.BlockSpec((br, 16), lambda i: (i, 0)),
            pl.BlockSpec((1, d), lambda i: (0, 0)),
            pl.BlockSpec((1, d), lambda i: (0, 0)),
            pl.BlockSpec(wpa.shape, lambda i: (0, 0)),
            pl.BlockSpec(wpb.shape, lambda i: (0, 0)),
            pl.BlockSpec(wga.shape, lambda i: (0, 0)),
            pl.BlockSpec(wgb.shape, lambda i: (0, 0)),
        ],
        out_specs=[
            pl.BlockSpec((br, wpa.shape[1]), lambda i: (i, 0)),
            pl.BlockSpec((br, wga.shape[1]), lambda i: (i, 0)),
            pl.BlockSpec((br, 1), lambda i: (i, 0)),
        ],
        out_shape=[
            jax.ShapeDtypeStruct((_N, wpa.shape[1]), _f32),
            jax.ShapeDtypeStruct((_N, wga.shape[1]), _f32),
            jax.ShapeDtypeStruct((_N, 1), _f32),
        ],
    )(ga, gb, aa, ab, cdeg, ba, bb, wpa, wpb, wga, wgb)


def _tc_node(g, a0, a1, dinv, b, wp, wg):
    br = 1000
    d = g.shape[1]

    def body(g_ref, a0_ref, a1_ref, dinv_ref, b_ref, wp_ref, wg_ref,
             p_ref, gn_ref):
        h = jnp.maximum(
            g_ref[...] + (a0_ref[...] + a1_ref[...]) * dinv_ref[...]
            + b_ref[...], 0.0)
        p_ref[...] = jnp.dot(h, wp_ref[...], preferred_element_type=_f32)
        gn_ref[...] = jnp.dot(h, wg_ref[...], preferred_element_type=_f32)

    return pl.pallas_call(
        body,
        grid=(_N // br,),
        in_specs=[
            pl.BlockSpec((br, d), lambda i: (i, 0)),
            pl.BlockSpec((br, d), lambda i: (i, 0)),
            pl.BlockSpec((br, d), lambda i: (i, 0)),
            pl.BlockSpec((br, 1), lambda i: (i, 0)),
            pl.BlockSpec((1, d), lambda i: (0, 0)),
            pl.BlockSpec(wp.shape, lambda i: (0, 0)),
            pl.BlockSpec(wg.shape, lambda i: (0, 0)),
        ],
        out_specs=[
            pl.BlockSpec((br, wp.shape[1]), lambda i: (i, 0)),
            pl.BlockSpec((br, wg.shape[1]), lambda i: (i, 0)),
        ],
        out_shape=[
            jax.ShapeDtypeStruct((_N, wp.shape[1]), _f32),
            jax.ShapeDtypeStruct((_N, wg.shape[1]), _f32),
        ],
    )(g, a0, a1, dinv, b, wp, wg)


def _tc_node_last(g, a0, a1, dinv, b, wp, shift):
    br = 1000
    d = g.shape[1]

    def body(g_ref, a0_ref, a1_ref, dinv_ref, b_ref, wp_ref, s_ref, p_ref):
        h = jnp.maximum(
            g_ref[...] + (a0_ref[...] + a1_ref[...]) * dinv_ref[...]
            + b_ref[...], 0.0)
        p_ref[...] = jnp.dot(h, wp_ref[...],
                             preferred_element_type=_f32) + s_ref[...]

    return pl.pallas_call(
        body,
        grid=(_N // br,),
        in_specs=[
            pl.BlockSpec((br, d), lambda i: (i, 0)),
            pl.BlockSpec((br, d), lambda i: (i, 0)),
            pl.BlockSpec((br, d), lambda i: (i, 0)),
            pl.BlockSpec((br, 1), lambda i: (i, 0)),
            pl.BlockSpec((1, d), lambda i: (0, 0)),
            pl.BlockSpec(wp.shape, lambda i: (0, 0)),
            pl.BlockSpec((1, 1), lambda i: (0, 0)),
        ],
        out_specs=pl.BlockSpec((br, 1), lambda i: (i, 0)),
        out_shape=jax.ShapeDtypeStruct((_N, 1), _f32),
    )(g, a0, a1, dinv, b, wp, shift)


def _tc_edge1(es, ed, bc, wf, br=8000):
    d = es.shape[1]

    def body(es_ref, ed_ref, bc_ref, wf_ref, t_ref):
        ce = jnp.maximum(es_ref[...] + ed_ref[...] + bc_ref[...], 0.0)
        t_ref[...] = jnp.dot(ce, wf_ref[...], preferred_element_type=_f32)

    return pl.pallas_call(
        body,
        grid=(_E // br,),
        in_specs=[
            pl.BlockSpec((br, d), lambda i: (i, 0)),
            pl.BlockSpec((br, d), lambda i: (i, 0)),
            pl.BlockSpec((1, d), lambda i: (0, 0)),
            pl.BlockSpec(wf.shape, lambda i: (0, 0)),
        ],
        out_specs=pl.BlockSpec((br, wf.shape[1]), lambda i: (i, 0)),
        out_shape=jax.ShapeDtypeStruct((_E, wf.shape[1]), _f32),
    )(es, ed, bc, wf)


def _tc_edge2(t1, e2s, e2d, bc2, bf1, wf2, bf2, br=8000):
    d = t1.shape[1]

    def body(t1_ref, es_ref, ed_ref, bc2_ref, bf1_ref, wf2_ref, bf2_ref,
             o_ref):
        ce2 = jnp.maximum(es_ref[...] + ed_ref[...] + bc2_ref[...], 0.0)
        z = t1_ref[...] + bf1_ref[...] + ce2
        f1 = 1.0 / (1.0 + jnp.exp(-z))
        o_ref[...] = jnp.dot(f1, wf2_ref[...],
                             preferred_element_type=_f32) + bf2_ref[...]

    return pl.pallas_call(
        body,
        grid=(_E // br,),
        in_specs=[
            pl.BlockSpec((br, d), lambda i: (i, 0)),
            pl.BlockSpec((br, d), lambda i: (i, 0)),
            pl.BlockSpec((br, d), lambda i: (i, 0)),
            pl.BlockSpec((1, d), lambda i: (0, 0)),
            pl.BlockSpec((1, d), lambda i: (0, 0)),
            pl.BlockSpec(wf2.shape, lambda i: (0, 0)),
            pl.BlockSpec((1, 1), lambda i: (0, 0)),
        ],
        out_specs=pl.BlockSpec((br, 1), lambda i: (i, 0)),
        out_shape=jax.ShapeDtypeStruct((_E, 1), _f32),
    )(t1, e2s, e2d, bc2, bf1, wf2, bf2)


def kernel(x, edge_index, W1, b1, Wc1, bc1, W2, b2, Wc2, bc2, W3, b3,
           Wc3, bc3, Wf1, bf1, Wf2, bf2):
    src = edge_index[0]
    dst = edge_index[1]
    src_t = src.reshape(_NW, _NCH, _C)
    dst_t = dst.reshape(_NW, _NCH, _C)
    src_s = src.reshape(_NS, _NCH1, _C1)
    dst_s = dst.reshape(_NS, _NCH1, _C1)
    z64 = jnp.zeros((_N, 64), _f32)
    z32 = jnp.zeros((_N, 32), _f32)
    z16 = jnp.zeros((_N, 16), _f32)
    ones16 = jnp.ones((_C1, 16), _f32)

    g1a = _tc_mm(x, W1[:, :64])
    g1b = _tc_mm(x, W1[:, 64:])
    a1p, deg = _sc_deg_seg(g1a, g1b, src_s, dst_s, z64, z16, ones16)
    p1, g2, dinv = _tc_node1(g1a, g1b, a1p[0], a1p[1], deg,
                             b1[:64].reshape(1, -1), b1[64:].reshape(1, -1),
                             Wc1[:64], Wc1[64:], W2[:64], W2[64:])
    e1s, e1d, a2p = _sc_cond_seg(p1, g2, src_t, dst_t, z64)
    p2, g3 = _tc_node(g2, a2p[0], a2p[1], dinv,
                      b2.reshape(1, -1), Wc2, W3)
    t1 = _tc_edge1(e1s, e1d, bc1.reshape(1, -1), Wf1)
    e2s, e2d, a3p = _sc_cond_seg(p2, g3, src_t, dst_t, z32)
    p3b = _tc_node_last(g3, a3p[0], a3p[1], dinv, b3.reshape(1, -1),
                        Wc3, (0.5 * bc3).reshape(1, 1))
    t2 = _tc_edge2(t1, e2s, e2d, bc2.reshape(1, -1), bf1.reshape(1, -1),
                   Wf2, bf2.reshape(1, 1))
    out = _sc_final(p3b.reshape(_N), t2.reshape(_E), src, dst)
    return out.reshape(_E, 1)
